# baseline (speedup 1.0000x reference)
import jax
import jax.numpy as jnp
from jax.experimental import pallas as pl
from jax.experimental.pallas import tpu as pltpu


def _erf(v):
    s = jnp.sign(v)
    av = jnp.abs(v)
    t = 1.0 / (1.0 + 0.3275911 * av)
    poly = t * (0.254829592 + t * (-0.284496736 + t * (1.421413741
           + t * (-1.453152027 + t * 1.061405429))))
    return s * (1.0 - poly * jnp.exp(-av * av))


def _gelu_exact(h):
    return 0.5 * h * (1.0 + _erf(h * 0.7071067811865476))


def _make_body(NCH, E, C):
    def body(dm_ref, cm_ref, x_ref, w1_ref, b1_ref, w2_ref, b2_ref,
             out_ref, d_scr, y_scr):
        i = pl.program_id(0)

        @pl.when(i < NCH)
        def _dispatch():
            contrib = jax.lax.dot_general(
                dm_ref[...], x_ref[...],
                dimension_numbers=(((0,), (0,)), ((), ())),
                preferred_element_type=jnp.float32,
            )

            @pl.when(i == 0)
            def _():
                d_scr[...] = contrib

            @pl.when(i != 0)
            def _():
                d_scr[...] += contrib

        @pl.when((i >= NCH) & (i < NCH + E))
        def _ffn():
            e = i - NCH
            de = d_scr[pl.ds(e * C, C), :]
            h = jnp.dot(de, w1_ref[0], preferred_element_type=jnp.float32)
            h = _gelu_exact(h + b1_ref[0])
            y = jnp.dot(h, w2_ref[0], preferred_element_type=jnp.float32)
            y_scr[pl.ds(e * C, C), :] = y + b2_ref[0]

        @pl.when(i >= NCH + E)
        def _combine():
            out_ref[...] = jnp.dot(
                cm_ref[...], y_scr[...], preferred_element_type=jnp.float32)

    return body


def kernel(x, dispatch_mask, combine_array, W1, b1, W2, b2):
    B, S, D = x.shape
    _, _, E, C = dispatch_mask.shape
    HE = W1.shape[2]
    EC = E * C

    Sb = 512
    NCH = S // Sb
    last = NCH - 1

    xs = x.reshape(S, D)
    dm = dispatch_mask.reshape(S, EC)
    cm = combine_array.reshape(S, EC)
    b1r = b1.reshape(E, 1, HE)
    b2r = b2.reshape(E, 1, D)

    grid = (NCH + E + NCH,)

    out = pl.pallas_call(
        _make_body(NCH, E, C),
        grid=grid,
        in_specs=[
            pl.BlockSpec((Sb, EC), lambda i: (jnp.minimum(i, last), 0)),
            pl.BlockSpec((Sb, EC), lambda i: (jnp.clip(i - (NCH + E), 0, last), 0)),
            pl.BlockSpec((Sb, D), lambda i: (jnp.minimum(i, last), 0)),
            pl.BlockSpec((1, D, HE), lambda i: (jnp.clip(i - NCH, 0, E - 1), 0, 0)),
            pl.BlockSpec((1, 1, HE), lambda i: (jnp.clip(i - NCH, 0, E - 1), 0, 0)),
            pl.BlockSpec((1, HE, D), lambda i: (jnp.clip(i - NCH, 0, E - 1), 0, 0)),
            pl.BlockSpec((1, 1, D), lambda i: (jnp.clip(i - NCH, 0, E - 1), 0, 0)),
        ],
        out_specs=pl.BlockSpec((Sb, D), lambda i: (jnp.clip(i - (NCH + E), 0, last), 0)),
        out_shape=jax.ShapeDtypeStruct((S, D), jnp.float32),
        scratch_shapes=[
            pltpu.VMEM((EC, D), jnp.float32),
            pltpu.VMEM((EC, D), jnp.float32),
        ],
    )(dm, cm, xs, W1, b1r, W2, b2r)
    return out.reshape(B, S, D)
